# baseline (speedup 1.0000x reference)
import functools

import jax
import jax.numpy as jnp
from jax import lax
from jax.experimental import pallas as pl
from jax.experimental.pallas import tpu as pltpu
from jax.experimental.pallas import tpu_sc as plsc

NUM_EMB = 1000000
DIM = 64
BATCH = 4096
HIST = 50

NC = 2
NS = 16
NW = NC * NS
CHUNK = 128
NCHUNK = HIST
NBUF = 5
NTBUF = 5


def _body(idx_hbm, table_hbm, out_hbm, idx_v, rows_v, tbuf, gsem, osem):
  wid = lax.axis_index("s") * NC + lax.axis_index("c")
  base_b = wid * CHUNK

  pltpu.sync_copy(idx_hbm.at[:, pl.ds(base_b, CHUNK)], idx_v)

  def start_gather(j, b):
    pltpu.async_copy(table_hbm.at[idx_v.at[j]], rows_v.at[b], gsem.at[b])

  def wait_gather(j, b):
    pltpu.make_async_copy(table_hbm.at[idx_v.at[j]], rows_v.at[b],
                          gsem.at[b]).wait()

  def start_writes(j, tb):
    for dhi in range(8):
      pltpu.async_copy(tbuf.at[tb, pl.ds(dhi * 8, 8), pl.ds(0, CHUNK)],
                       out_hbm.at[j, dhi, wid], osem.at[tb])

  def wait_writes(j, tb):
    for dhi in range(8):
      pltpu.make_async_copy(tbuf.at[tb, pl.ds(dhi * 8, 8), pl.ds(0, CHUNK)],
                            out_hbm.at[j, dhi, wid], osem.at[tb]).wait()

  lanes = lax.iota(jnp.int32, 16)
  dvecs = [lanes + (16 * q) for q in range(4)]

  def transpose_chunk(b, tb):
    @pl.loop(0, CHUNK, step=8)
    def _l(l0):
      for dl in range(8):
        l = l0 + dl
        col = jnp.broadcast_to(l, (16,)).astype(jnp.int32)
        for q in range(4):
          v = rows_v[b, l, pl.ds(16 * q, 16)]
          plsc.store_scatter(tbuf.at[tb], [dvecs[q], col], v)

  for b in range(NBUF):
    start_gather(b, b)

  @pl.loop(0, NCHUNK, step=NBUF)
  def _outer(j0):
    for b in range(NBUF):
      j = j0 + b
      bp = (b - 1) % NBUF
      tb = b
      wait_gather(j, b)

      if b == 0:
        @pl.when((j >= 1) & (j + NBUF - 1 < NCHUNK))
        def _():
          start_gather(j + NBUF - 1, bp)
      else:
        @pl.when(j + NBUF - 1 < NCHUNK)
        def _():
          start_gather(j + NBUF - 1, bp)

      @pl.when(j >= NTBUF)
      def _():
        wait_writes(j - NTBUF, tb)

      transpose_chunk(b, tb)
      start_writes(j, tb)

  for b in range(NTBUF):
    wait_writes(NCHUNK - NTBUF + b, b)


@jax.jit
def kernel(indices, table):
  idx_t = indices.T.astype(jnp.int32)
  run = pl.kernel(
      _body,
      out_type=jax.ShapeDtypeStruct((HIST, 8, NW, 8, CHUNK), jnp.float32),
      mesh=plsc.VectorSubcoreMesh(core_axis_name="c", subcore_axis_name="s"),
      compiler_params=pltpu.CompilerParams(use_tc_tiling_on_sc=False,
                                           needs_layout_passes=False,
                                           disable_bounds_checks=True),
      scratch_types=[
          pltpu.VMEM((NCHUNK, CHUNK), jnp.int32),
          pltpu.VMEM((NBUF, CHUNK, DIM), jnp.float32),
          pltpu.VMEM((NTBUF, DIM, CHUNK + 1), jnp.float32),
          pltpu.SemaphoreType.DMA((NBUF,)),
          pltpu.SemaphoreType.DMA((NTBUF,)),
      ],
  )
  y = run(idx_t, table)
  return y.transpose(2, 4, 0, 1, 3).reshape(BATCH, HIST, DIM)


# speedup vs baseline: 1.0022x; 1.0022x over previous
import functools

import jax
import jax.numpy as jnp
from jax import lax
from jax.experimental import pallas as pl
from jax.experimental.pallas import tpu as pltpu
from jax.experimental.pallas import tpu_sc as plsc

NUM_EMB = 1000000
DIM = 64
BATCH = 4096
HIST = 50

NC = 2
NS = 16
NW = NC * NS
CHUNK = 128
NCHUNK = HIST
NBUF = 5
NTBUF = 5


def _body(idx_hbm, table_hbm, out_hbm, idx_v, rows_v, tbuf, gsem, osem):
  wid = lax.axis_index("s") * NC + lax.axis_index("c")
  base_b = wid * CHUNK

  pltpu.sync_copy(idx_hbm.at[:, pl.ds(base_b, CHUNK)], idx_v)

  def start_gather(j, b):
    pltpu.async_copy(table_hbm.at[idx_v.at[j]], rows_v.at[b], gsem.at[b])

  def wait_gather(j, b):
    pltpu.make_async_copy(table_hbm.at[idx_v.at[j]], rows_v.at[b],
                          gsem.at[b]).wait()

  def start_writes(j, tb):
    for dhi in range(8):
      pltpu.async_copy(tbuf.at[tb, pl.ds(dhi * 8, 8), pl.ds(0, CHUNK)],
                       out_hbm.at[j, dhi, wid], osem.at[tb])

  def wait_writes(j, tb):
    for dhi in range(8):
      pltpu.make_async_copy(tbuf.at[tb, pl.ds(dhi * 8, 8), pl.ds(0, CHUNK)],
                            out_hbm.at[j, dhi, wid], osem.at[tb]).wait()

  lanes = lax.iota(jnp.int32, 16)
  dvecs = [lanes + (16 * q) for q in range(4)]

  def transpose_chunk(b, tb):
    @pl.loop(0, CHUNK, step=2)
    def _l(l0):
      for dl in range(2):
        l = l0 + dl
        col = jnp.broadcast_to(l, (16,)).astype(jnp.int32)
        for q in range(4):
          v = rows_v[b, l, pl.ds(16 * q, 16)]
          plsc.store_scatter(tbuf.at[tb], [dvecs[q], col], v)

  for b in range(NBUF):
    start_gather(b, b)

  @pl.loop(0, NCHUNK, step=NBUF)
  def _outer(j0):
    for b in range(NBUF):
      j = j0 + b
      bp = (b - 1) % NBUF
      tb = b
      wait_gather(j, b)

      if b == 0:
        @pl.when((j >= 1) & (j + NBUF - 1 < NCHUNK))
        def _():
          start_gather(j + NBUF - 1, bp)
      else:
        @pl.when(j + NBUF - 1 < NCHUNK)
        def _():
          start_gather(j + NBUF - 1, bp)

      @pl.when(j >= NTBUF)
      def _():
        wait_writes(j - NTBUF, tb)

      transpose_chunk(b, tb)
      start_writes(j, tb)

  for b in range(NTBUF):
    wait_writes(NCHUNK - NTBUF + b, b)


@jax.jit
def kernel(indices, table):
  idx_t = indices.T.astype(jnp.int32)
  run = pl.kernel(
      _body,
      out_type=jax.ShapeDtypeStruct((HIST, 8, NW, 8, CHUNK), jnp.float32),
      mesh=plsc.VectorSubcoreMesh(core_axis_name="c", subcore_axis_name="s"),
      compiler_params=pltpu.CompilerParams(use_tc_tiling_on_sc=False,
                                           needs_layout_passes=False,
                                           disable_bounds_checks=True),
      scratch_types=[
          pltpu.VMEM((NCHUNK, CHUNK), jnp.int32),
          pltpu.VMEM((NBUF, CHUNK, DIM), jnp.float32),
          pltpu.VMEM((NTBUF, DIM, CHUNK + 1), jnp.float32),
          pltpu.SemaphoreType.DMA((NBUF,)),
          pltpu.SemaphoreType.DMA((NTBUF,)),
      ],
  )
  y = run(idx_t, table)
  return y.transpose(2, 4, 0, 1, 3).reshape(BATCH, HIST, DIM)
